# Initial kernel scaffold; baseline (speedup 1.0000x reference)
import jax
import jax.numpy as jnp
from jax.experimental import pallas as pl


def kernel(x, edge_index_rel0, edge_index_rel1, W_rel0, b_rel0, W_rel1, b_rel1, W_self, b_self, W_neigh, b_neigh):
    raise NotImplementedError("write your pallas kernel here")


# speedup vs baseline: 3.7735x; 3.7735x over previous
import functools

import jax
import jax.numpy as jnp
from jax import lax
from jax.experimental import pallas as pl
from jax.experimental.pallas import tpu as pltpu
from jax.experimental.pallas import tpu_sc as plsc

_CH = 80
_GB = 25
_NSUB = 16
_NCORE = 2


def _sc_segment_sums(x, src1d, dst1d, z_acc, ones_rows, n_pad):
    n, d = x.shape
    epw = src1d.shape[0] // (_NCORE * _NSUB)
    nchunks = epw // _CH
    stripe = n_pad // _NSUB

    mesh = plsc.VectorSubcoreMesh(core_axis_name="c", subcore_axis_name="s")

    @functools.partial(
        pl.kernel,
        out_type=[
            jax.ShapeDtypeStruct((_NCORE, n_pad, d), jnp.float32),
            jax.ShapeDtypeStruct((_NCORE, n_pad, d), jnp.float32),
        ],
        mesh=mesh,
        scratch_types=[
            pltpu.VMEM((_CH,), jnp.int32),
            pltpu.VMEM((_CH,), jnp.int32),
            pltpu.VMEM((_CH, d), jnp.float32),
            pltpu.VMEM_SHARED((n_pad, d), jnp.float32),
            pltpu.SemaphoreType.DMA,
        ],
    )
    def seg(x_hbm, src_hbm, dst_hbm, zacc_hbm, ones_hbm,
            s_hbm, c_hbm, srcv, dstv, rows, acc, sem):
        cid = lax.axis_index("c")
        sid = lax.axis_index("s")
        wid = cid * _NSUB + sid
        r0 = sid * stripe
        base0 = wid * epw

        pltpu.sync_copy(zacc_hbm, acc.at[pl.ds(r0, stripe)])
        plsc.subcore_barrier()

        @pl.loop(0, nchunks)
        def _(j):
            pltpu.sync_copy(src_hbm.at[pl.ds(base0 + j * _CH, _CH)], srcv)
            pltpu.sync_copy(dst_hbm.at[pl.ds(base0 + j * _CH, _CH)], dstv)
            pltpu.async_copy(x_hbm.at[srcv], rows, sem).wait()
            pltpu.sync_copy(rows, acc.at[dstv], add=True)

        plsc.subcore_barrier()
        pltpu.sync_copy(acc.at[pl.ds(r0, stripe)], s_hbm.at[cid, pl.ds(r0, stripe)])

        pltpu.sync_copy(zacc_hbm, acc.at[pl.ds(r0, stripe)])
        pltpu.sync_copy(ones_hbm, rows)
        plsc.subcore_barrier()

        @pl.loop(0, nchunks)
        def _(j):
            pltpu.sync_copy(dst_hbm.at[pl.ds(base0 + j * _CH, _CH)], dstv)
            pltpu.sync_copy(rows, acc.at[dstv], add=True)

        plsc.subcore_barrier()
        pltpu.sync_copy(acc.at[pl.ds(r0, stripe)], c_hbm.at[cid, pl.ds(r0, stripe)])

    return seg(x, src1d, dst1d, z_acc, ones_rows)


def _combine_body(x_ref, s0_ref, s1_ref, c0_ref, c1_ref,
                  w0_ref, w1_ref, ws_ref, wn_ref,
                  b0_ref, b1_ref, bc_ref, o_ref):
    c0 = c0_ref[0, :, 0:1]
    c1 = c1_ref[0, :, 0:1]
    m0 = s0_ref[0] / jnp.maximum(c0, 1.0)
    m1 = s1_ref[0] / jnp.maximum(c1, 1.0)
    e0 = (c0 > 0.0).astype(jnp.float32)
    e1 = (c1 > 0.0).astype(jnp.float32)
    h = (jnp.dot(m0, w0_ref[...], preferred_element_type=jnp.float32)
         + jnp.dot(m1, w1_ref[...], preferred_element_type=jnp.float32)
         + e0 * b0_ref[...] + e1 * b1_ref[...])
    o_ref[...] = (jnp.dot(x_ref[...], ws_ref[...], preferred_element_type=jnp.float32)
                  + jnp.dot(h, wn_ref[...], preferred_element_type=jnp.float32)
                  + bc_ref[...])


def _combine(x, s, c, w0, w1, ws, wn, b0r, b1r, bcr):
    n, d = x.shape
    bn_rows = 1000
    nb = n // bn_rows
    full = pl.BlockSpec((d, d), lambda i: (0, 0))
    brow = pl.BlockSpec((1, d), lambda i: (0, 0))
    return pl.pallas_call(
        _combine_body,
        grid=(nb,),
        in_specs=[
            pl.BlockSpec((bn_rows, d), lambda i: (i, 0)),
            pl.BlockSpec((1, bn_rows, d), lambda i: (0, i, 0)),
            pl.BlockSpec((1, bn_rows, d), lambda i: (1, i, 0)),
            pl.BlockSpec((1, bn_rows, d), lambda i: (0, i, 0)),
            pl.BlockSpec((1, bn_rows, d), lambda i: (1, i, 0)),
            full, full, full, full,
            brow, brow, brow,
        ],
        out_specs=pl.BlockSpec((bn_rows, d), lambda i: (i, 0)),
        out_shape=jax.ShapeDtypeStruct((n, d), jnp.float32),
    )(x, s, s, c, c, w0, w1, ws, wn, b0r, b1r, bcr)


def kernel(x, edge_index_rel0, edge_index_rel1, W_rel0, b_rel0, W_rel1, b_rel1,
           W_self, b_self, W_neigh, b_neigh):
    n, d = x.shape
    src1d = jnp.concatenate(
        [edge_index_rel0[0], edge_index_rel1[0]]).astype(jnp.int32)
    dst1d = jnp.concatenate(
        [edge_index_rel0[1], edge_index_rel1[1]]).astype(jnp.int32)
    n_pad = ((n + 8 * _NSUB - 1) // (8 * _NSUB)) * (8 * _NSUB)
    stripe = n_pad // _NSUB
    z_acc = jnp.zeros((stripe, d), jnp.float32)
    ones_rows = jnp.ones((_CH, d), jnp.float32)
    s, c = _sc_segment_sums(x, src1d, dst1d, z_acc, ones_rows, n_pad)
    b0r = b_rel0.reshape(1, d)
    b1r = b_rel1.reshape(1, d)
    bcr = (b_self + b_neigh).reshape(1, d)
    return _combine(x, s, c, W_rel0, W_rel1, W_self, W_neigh, b0r, b1r, bcr)


# speedup vs baseline: 4.8556x; 1.2868x over previous
import functools

import jax
import jax.numpy as jnp
from jax import lax
from jax.experimental import pallas as pl
from jax.experimental.pallas import tpu as pltpu
from jax.experimental.pallas import tpu_sc as plsc

_CH = 40
_GB = 10
_NSUB = 16
_NCORE = 2
_RB = 64


def _sc_segment_sums(x, src1d, dst1d, z_acc, n_pad):
    n, d = x.shape
    nw = _NCORE * _NSUB
    epw = src1d.shape[0] // nw
    ngroups = epw // (_GB * _CH)
    stripe = n_pad // _NSUB
    ch2 = 2 * _CH
    g2 = 25
    ng2 = epw // (g2 * ch2)

    src4d = src1d.reshape(nw, ngroups, _GB, _CH)
    dst4d = dst1d.reshape(nw, ngroups, _GB, _CH)
    dst4d2 = dst1d.reshape(nw, ng2, g2, ch2)

    mesh = plsc.VectorSubcoreMesh(core_axis_name="c", subcore_axis_name="s")

    @functools.partial(
        pl.kernel,
        out_type=[
            jax.ShapeDtypeStruct((_NCORE, n_pad, d), jnp.float32),
            jax.ShapeDtypeStruct((_NCORE, n_pad, d), jnp.float32),
        ],
        mesh=mesh,
        scratch_types=[
            pltpu.VMEM((2, _GB, _CH), jnp.int32),
            pltpu.VMEM((g2, ch2), jnp.int32),
            pltpu.VMEM((2 * _CH, d), jnp.float32),
            pltpu.VMEM_SHARED((n_pad, d), jnp.float32),
            pltpu.SemaphoreType.DMA,
            pltpu.SemaphoreType.DMA,
        ],
    )
    def seg(x_hbm, src_hbm, dst_hbm, dst2_hbm, zacc_hbm, ones_hbm,
            s_hbm, c_hbm, idxv, dstv2, rowsb, acc, sem0, sem1):
        cid = lax.axis_index("c")
        sid = lax.axis_index("s")
        wid = cid * _NSUB + sid
        r0 = sid * stripe

        pltpu.sync_copy(zacc_hbm, acc.at[pl.ds(r0, stripe)])
        plsc.subcore_barrier()

        srcv = idxv.at[0]
        dstv = idxv.at[1]
        rows0 = rowsb.at[pl.ds(0, _CH)]
        rows1 = rowsb.at[pl.ds(_CH, _CH)]

        @pl.loop(0, ngroups)
        def _(g):
            pltpu.sync_copy(src_hbm.at[wid, g], srcv)
            pltpu.sync_copy(dst_hbm.at[wid, g], dstv)

            @pl.loop(0, _GB // 2)
            def _(t):
                j0 = 2 * t
                j1 = 2 * t + 1
                cp0 = pltpu.async_copy(x_hbm.at[srcv.at[j0]], rows0, sem0)
                cp1 = pltpu.async_copy(x_hbm.at[srcv.at[j1]], rows1, sem1)
                cp0.wait()
                pltpu.sync_copy(rows0, acc.at[dstv.at[j0]], add=True)
                cp1.wait()
                pltpu.sync_copy(rows1, acc.at[dstv.at[j1]], add=True)

        plsc.subcore_barrier()
        pltpu.sync_copy(acc.at[pl.ds(r0, stripe)], s_hbm.at[cid, pl.ds(r0, stripe)])

        pltpu.sync_copy(zacc_hbm, acc.at[pl.ds(r0, stripe)])
        pltpu.sync_copy(ones_hbm, rowsb)
        plsc.subcore_barrier()

        @pl.loop(0, ng2)
        def _(g):
            pltpu.sync_copy(dst2_hbm.at[wid, g], dstv2)

            @pl.loop(0, g2)
            def _(k):
                pltpu.sync_copy(rowsb, acc.at[dstv2.at[k]], add=True)

        plsc.subcore_barrier()
        pltpu.sync_copy(acc.at[pl.ds(r0, stripe)], c_hbm.at[cid, pl.ds(r0, stripe)])

    ones_rows = jnp.ones((ch2, d), jnp.float32)
    return seg(x, src4d, dst4d, dst4d2, z_acc, ones_rows)


def _combine_body(x_ref, s0_ref, s1_ref, c0_ref, c1_ref,
                  w0_ref, w1_ref, ws_ref, wn_ref,
                  b0_ref, b1_ref, bc_ref, o_ref):
    c0 = c0_ref[0, :, 0:1]
    c1 = c1_ref[0, :, 0:1]
    m0 = s0_ref[0] / jnp.maximum(c0, 1.0)
    m1 = s1_ref[0] / jnp.maximum(c1, 1.0)
    e0 = (c0 > 0.0).astype(jnp.float32)
    e1 = (c1 > 0.0).astype(jnp.float32)
    h = (jnp.dot(m0, w0_ref[...], preferred_element_type=jnp.float32)
         + jnp.dot(m1, w1_ref[...], preferred_element_type=jnp.float32)
         + e0 * b0_ref[...] + e1 * b1_ref[...])
    o_ref[...] = (jnp.dot(x_ref[...], ws_ref[...], preferred_element_type=jnp.float32)
                  + jnp.dot(h, wn_ref[...], preferred_element_type=jnp.float32)
                  + bc_ref[...])


def _combine(x, s, c, w0, w1, ws, wn, b0r, b1r, bcr):
    n, d = x.shape
    bn_rows = 1000
    nb = n // bn_rows
    full = pl.BlockSpec((d, d), lambda i: (0, 0))
    brow = pl.BlockSpec((1, d), lambda i: (0, 0))
    return pl.pallas_call(
        _combine_body,
        grid=(nb,),
        in_specs=[
            pl.BlockSpec((bn_rows, d), lambda i: (i, 0)),
            pl.BlockSpec((1, bn_rows, d), lambda i: (0, i, 0)),
            pl.BlockSpec((1, bn_rows, d), lambda i: (1, i, 0)),
            pl.BlockSpec((1, bn_rows, d), lambda i: (0, i, 0)),
            pl.BlockSpec((1, bn_rows, d), lambda i: (1, i, 0)),
            full, full, full, full,
            brow, brow, brow,
        ],
        out_specs=pl.BlockSpec((bn_rows, d), lambda i: (i, 0)),
        out_shape=jax.ShapeDtypeStruct((n, d), jnp.float32),
    )(x, s, s, c, c, w0, w1, ws, wn, b0r, b1r, bcr)


def kernel(x, edge_index_rel0, edge_index_rel1, W_rel0, b_rel0, W_rel1, b_rel1,
           W_self, b_self, W_neigh, b_neigh):
    n, d = x.shape
    src1d = jnp.concatenate(
        [edge_index_rel0[0], edge_index_rel1[0]]).astype(jnp.int32)
    dst1d = jnp.concatenate(
        [edge_index_rel0[1], edge_index_rel1[1]]).astype(jnp.int32)
    n_pad = ((n + 64 * _NSUB - 1) // (64 * _NSUB)) * (64 * _NSUB)
    stripe = n_pad // _NSUB
    z_acc = jnp.zeros((stripe, d), jnp.float32)
    s, c = _sc_segment_sums(x, src1d, dst1d, z_acc, n_pad)
    b0r = b_rel0.reshape(1, d)
    b1r = b_rel1.reshape(1, d)
    bcr = (b_self + b_neigh).reshape(1, d)
    return _combine(x, s, c, W_rel0, W_rel1, W_self, W_neigh, b0r, b1r, bcr)
